# Initial kernel scaffold; baseline (speedup 1.0000x reference)
import jax
import jax.numpy as jnp
from jax.experimental import pallas as pl


def kernel(x, edge_index, W1, b1, W2, b2):
    raise NotImplementedError("write your pallas kernel here")


# speedup vs baseline: 29.3077x; 29.3077x over previous
import functools

import jax
import jax.numpy as jnp
from jax import lax
from jax.experimental import pallas as pl
from jax.experimental.pallas import tpu as pltpu
from jax.experimental.pallas import tpu_sc as plsc

N = 10000
E = 320000
F_IN = 128
HID = 16
CLS = 40

NC = 2
NS = 16
L = 16

NW = NC * NS
EPW = E // NW
CH = 80
NCH = EPW // CH
NPAD = 10240
NPS = NPAD // NS
WB = 128
NWB = NPS // WB

_mesh = plsc.VectorSubcoreMesh(core_axis_name="c", subcore_axis_name="s")

_f32 = jnp.float32


def _zero_fill(buf, nrows):
    zero = jnp.zeros((L,), _f32)

    def fill(i, carry):
        buf[i, :] = zero
        return carry

    lax.fori_loop(0, nrows, fill, 0)


@functools.partial(
    pl.kernel,
    out_type=jax.ShapeDtypeStruct((NC, NPAD, HID), _f32),
    mesh=_mesh,
    compiler_params=pltpu.CompilerParams(use_tc_tiling_on_sc=False),
    scratch_types=[
        pltpu.VMEM((NCH, CH), jnp.int32),
        pltpu.VMEM((CH, HID), _f32),
        pltpu.VMEM((WB, HID), _f32),
        pltpu.VMEM_SHARED((NPAD, HID), _f32),
    ],
)
def _sc_deg(dst_hbm, out_hbm, didx, ones_b, buf, acc):
    c = lax.axis_index("c")
    s = lax.axis_index("s")
    wid = c * NS + s
    pltpu.sync_copy(dst_hbm.at[wid], didx)

    _zero_fill(buf, WB)
    one = jnp.ones((L,), _f32)

    def fill_ones(i, carry):
        ones_b[i, :] = one
        return carry

    lax.fori_loop(0, CH, fill_ones, 0)

    for k in range(NWB):
        pltpu.sync_copy(buf, acc.at[pl.ds(s * NPS + k * WB, WB)])
    plsc.subcore_barrier()

    def body(j, carry):
        pltpu.sync_copy(ones_b, acc.at[didx.at[j]], add=True)
        return carry

    lax.fori_loop(0, NCH, body, 0)
    plsc.subcore_barrier()

    for k in range(NWB):
        r0 = s * NPS + k * WB
        pltpu.sync_copy(acc.at[pl.ds(r0, WB)], buf)
        pltpu.sync_copy(buf, out_hbm.at[c].at[pl.ds(r0, WB)])


@functools.partial(
    pl.kernel,
    out_type=jax.ShapeDtypeStruct((NC, NPAD, HID), _f32),
    mesh=_mesh,
    compiler_params=pltpu.CompilerParams(use_tc_tiling_on_sc=False),
    scratch_types=[
        pltpu.VMEM((NCH, CH), jnp.int32),
        pltpu.VMEM((NCH, CH), jnp.int32),
        pltpu.VMEM((CH, HID), _f32),
        pltpu.VMEM((WB, HID), _f32),
        pltpu.VMEM_SHARED((NPAD, HID), _f32),
        pltpu.SemaphoreType.DMA,
    ],
)
def _sc_agg(g_hbm, src_hbm, dst_hbm, out_hbm, sidx, didx, rows, buf, acc, sem):
    c = lax.axis_index("c")
    s = lax.axis_index("s")
    wid = c * NS + s
    pltpu.sync_copy(src_hbm.at[wid], sidx)
    pltpu.sync_copy(dst_hbm.at[wid], didx)

    _zero_fill(buf, WB)
    for k in range(NWB):
        pltpu.sync_copy(buf, acc.at[pl.ds(s * NPS + k * WB, WB)])
    plsc.subcore_barrier()

    def body(j, carry):
        pltpu.async_copy(g_hbm.at[sidx.at[j]], rows, sem).wait()
        pltpu.sync_copy(rows, acc.at[didx.at[j]], add=True)
        return carry

    lax.fori_loop(0, NCH, body, 0)
    plsc.subcore_barrier()

    for k in range(NWB):
        r0 = s * NPS + k * WB
        pltpu.sync_copy(acc.at[pl.ds(r0, WB)], buf)
        pltpu.sync_copy(buf, out_hbm.at[c].at[pl.ds(r0, WB)])


def _tc_g1_body(x_ref, w1_ref, degp_ref, dis_ref, g1_ref):
    dis = lax.rsqrt(degp_ref[0, :N] + degp_ref[1, :N] + 1.0)
    h = jnp.dot(x_ref[...], w1_ref[...], preferred_element_type=_f32)
    dis_ref[...] = dis
    g1_ref[...] = h * dis


def _tc_g2_body(aggp_ref, g1_ref, dis_ref, b1_ref, g2_ref):
    dis = dis_ref[...]
    t = dis * (aggp_ref[0, :N] + aggp_ref[1, :N] + g1_ref[...]) + b1_ref[...]
    g2_ref[...] = jnp.maximum(t, 0.0) * dis


def _tc_out_body(aggp_ref, g2_ref, dis_ref, w2_ref, b2_ref, o_ref):
    t = dis_ref[...] * (aggp_ref[0, :N] + aggp_ref[1, :N] + g2_ref[...])
    o_ref[...] = jnp.dot(t, w2_ref[...], preferred_element_type=_f32) + b2_ref[...]


def kernel(x, edge_index, W1, b1, W2, b2):
    src3 = edge_index[0].reshape(NW, NCH, CH)
    dst3 = edge_index[1].reshape(NW, NCH, CH)

    degp = _sc_deg(dst3)

    dis_b, g1 = pl.pallas_call(
        _tc_g1_body,
        out_shape=[
            jax.ShapeDtypeStruct((N, HID), _f32),
            jax.ShapeDtypeStruct((N, HID), _f32),
        ],
    )(x, W1, degp)

    aggp1 = _sc_agg(g1, src3, dst3)

    g2 = pl.pallas_call(
        _tc_g2_body,
        out_shape=jax.ShapeDtypeStruct((N, HID), _f32),
    )(aggp1, g1, dis_b, b1.reshape(1, HID))

    aggp2 = _sc_agg(g2, src3, dst3)

    out = pl.pallas_call(
        _tc_out_body,
        out_shape=jax.ShapeDtypeStruct((N, CLS), _f32),
    )(aggp2, g2, dis_b, W2, b2.reshape(1, CLS))

    return out


# speedup vs baseline: 55.2397x; 1.8848x over previous
import functools

import jax
import jax.numpy as jnp
from jax import lax
from jax.experimental import pallas as pl
from jax.experimental.pallas import tpu as pltpu
from jax.experimental.pallas import tpu_sc as plsc

N = 10000
E = 320000
F_IN = 128
HID = 16
CLS = 40

NC = 2
NS = 16
L = 16

NW = NC * NS
EPW = E // NW
CH = 80
NCH = EPW // CH
NPAD = 10240
NPS = NPAD // NS
WB = 128
NWB = NPS // WB

_mesh = plsc.VectorSubcoreMesh(core_axis_name="c", subcore_axis_name="s")

_f32 = jnp.float32


def _zero_fill(buf, nrows):
    zero = jnp.zeros((L,), _f32)

    def fill(i, carry):
        buf[i, :] = zero
        return carry

    lax.fori_loop(0, nrows, fill, 0)


@functools.partial(
    pl.kernel,
    out_type=jax.ShapeDtypeStruct((NC, NPAD, HID), _f32),
    mesh=_mesh,
    compiler_params=pltpu.CompilerParams(use_tc_tiling_on_sc=False),
    scratch_types=[
        pltpu.VMEM((NCH, CH), jnp.int32),
        pltpu.VMEM((CH, HID), _f32),
        pltpu.VMEM((WB, HID), _f32),
        pltpu.VMEM_SHARED((NPAD, HID), _f32),
    ],
)
def _sc_deg(dst_hbm, out_hbm, didx, ones_b, buf, acc):
    c = lax.axis_index("c")
    s = lax.axis_index("s")
    wid = c * NS + s
    pltpu.sync_copy(dst_hbm.at[wid], didx)

    _zero_fill(buf, WB)
    one = jnp.ones((L,), _f32)

    def fill_ones(i, carry):
        ones_b[i, :] = one
        return carry

    lax.fori_loop(0, CH, fill_ones, 0)

    for k in range(NWB):
        pltpu.sync_copy(buf, acc.at[pl.ds(s * NPS + k * WB, WB)])
    plsc.subcore_barrier()

    def body(j, carry):
        pltpu.sync_copy(ones_b, acc.at[didx.at[j]], add=True)
        return carry

    lax.fori_loop(0, NCH, body, 0)
    plsc.subcore_barrier()

    for k in range(NWB):
        r0 = s * NPS + k * WB
        pltpu.sync_copy(acc.at[pl.ds(r0, WB)], buf)
        pltpu.sync_copy(buf, out_hbm.at[c].at[pl.ds(r0, WB)])


@functools.partial(
    pl.kernel,
    out_type=jax.ShapeDtypeStruct((NC, NPAD, HID), _f32),
    mesh=_mesh,
    compiler_params=pltpu.CompilerParams(use_tc_tiling_on_sc=False),
    scratch_types=[
        pltpu.VMEM((NCH, CH), jnp.int32),
        pltpu.VMEM((NCH, CH), jnp.int32),
        pltpu.VMEM((CH, HID), _f32),
        pltpu.VMEM((CH, HID), _f32),
        pltpu.VMEM((N // NS, HID), _f32),
        pltpu.VMEM((WB, HID), _f32),
        pltpu.VMEM_SHARED((N, HID), _f32),
        pltpu.VMEM_SHARED((NPAD, HID), _f32),
        pltpu.SemaphoreType.DMA,
        pltpu.SemaphoreType.DMA,
    ],
)
def _sc_agg(g_hbm, src_hbm, dst_hbm, out_hbm, sidx, didx, ra, rb, stg, buf,
            gs, acc, sa, sb):
    c = lax.axis_index("c")
    s = lax.axis_index("s")
    wid = c * NS + s
    pltpu.sync_copy(src_hbm.at[wid], sidx)
    pltpu.sync_copy(dst_hbm.at[wid], didx)

    t0 = s * (N // NS)
    pltpu.sync_copy(g_hbm.at[pl.ds(t0, N // NS)], stg)
    pltpu.sync_copy(stg, gs.at[pl.ds(t0, N // NS)])

    _zero_fill(buf, WB)
    for k in range(NWB):
        pltpu.sync_copy(buf, acc.at[pl.ds(s * NPS + k * WB, WB)])
    plsc.subcore_barrier()

    def start(j, r, sem):
        pltpu.async_copy(gs.at[sidx.at[j]], r, sem)

    def wait(j, r, sem):
        pltpu.make_async_copy(gs.at[sidx.at[j]], r, sem).wait()

    def scat(j, r):
        pltpu.sync_copy(r, acc.at[didx.at[j]], add=True)

    start(0, ra, sa)

    def body(p, carry):
        j0 = 2 * p
        j1 = j0 + 1
        start(j1, rb, sb)
        wait(j0, ra, sa)
        scat(j0, ra)
        start(j0 + 2, ra, sa)
        wait(j1, rb, sb)
        scat(j1, rb)
        return carry

    lax.fori_loop(0, (NCH - 1) // 2, body, 0)
    wait(NCH - 1, ra, sa)
    scat(NCH - 1, ra)
    plsc.subcore_barrier()

    for k in range(NWB):
        r0 = s * NPS + k * WB
        pltpu.sync_copy(acc.at[pl.ds(r0, WB)], buf)
        pltpu.sync_copy(buf, out_hbm.at[c].at[pl.ds(r0, WB)])


def _tc_g1_body(x_ref, w1_ref, degp_ref, dis_ref, g1_ref):
    dis = lax.rsqrt(degp_ref[0, :N] + degp_ref[1, :N] + 1.0)
    h = jnp.dot(x_ref[...], w1_ref[...], preferred_element_type=_f32)
    dis_ref[...] = dis
    g1_ref[...] = h * dis


def _tc_g2_body(aggp_ref, g1_ref, dis_ref, b1_ref, g2_ref):
    dis = dis_ref[...]
    t = dis * (aggp_ref[0, :N] + aggp_ref[1, :N] + g1_ref[...]) + b1_ref[...]
    g2_ref[...] = jnp.maximum(t, 0.0) * dis


def _tc_out_body(aggp_ref, g2_ref, dis_ref, w2_ref, b2_ref, o_ref):
    t = dis_ref[...] * (aggp_ref[0, :N] + aggp_ref[1, :N] + g2_ref[...])
    o_ref[...] = jnp.dot(t, w2_ref[...], preferred_element_type=_f32) + b2_ref[...]


def kernel(x, edge_index, W1, b1, W2, b2):
    src3 = edge_index[0].reshape(NW, NCH, CH)
    dst3 = edge_index[1].reshape(NW, NCH, CH)

    degp = _sc_deg(dst3)

    dis_b, g1 = pl.pallas_call(
        _tc_g1_body,
        out_shape=[
            jax.ShapeDtypeStruct((N, HID), _f32),
            jax.ShapeDtypeStruct((N, HID), _f32),
        ],
    )(x, W1, degp)

    aggp1 = _sc_agg(g1, src3, dst3)

    g2 = pl.pallas_call(
        _tc_g2_body,
        out_shape=jax.ShapeDtypeStruct((N, HID), _f32),
    )(aggp1, g1, dis_b, b1.reshape(1, HID))

    aggp2 = _sc_agg(g2, src3, dst3)

    out = pl.pallas_call(
        _tc_out_body,
        out_shape=jax.ShapeDtypeStruct((N, CLS), _f32),
    )(aggp2, g2, dis_b, W2, b2.reshape(1, CLS))

    return out


# speedup vs baseline: 77.8538x; 1.4094x over previous
import functools

import jax
import jax.numpy as jnp
from jax import lax
from jax.experimental import pallas as pl
from jax.experimental.pallas import tpu as pltpu
from jax.experimental.pallas import tpu_sc as plsc

N = 10000
E = 320000
F_IN = 128
HID = 16
CLS = 40

NC = 2
NS = 16
L = 16

NW = NC * NS
EPW = E // NW
CH = 80
NCH = EPW // CH
NPAD = 10240
NPS = NPAD // NS
WB = 128
NWB = NPS // WB

_mesh = plsc.VectorSubcoreMesh(core_axis_name="c", subcore_axis_name="s")

_f32 = jnp.float32


def _zero_fill(buf, nrows):
    zero = jnp.zeros((L,), _f32)

    def fill(i, carry):
        buf[i, :] = zero
        return carry

    lax.fori_loop(0, nrows, fill, 0)


@functools.partial(
    pl.kernel,
    out_type=jax.ShapeDtypeStruct((NC, NPAD, HID), _f32),
    mesh=_mesh,
    compiler_params=pltpu.CompilerParams(use_tc_tiling_on_sc=False),
    scratch_types=[
        pltpu.VMEM((NCH, CH), jnp.int32),
        pltpu.VMEM((CH, HID), _f32),
        pltpu.VMEM((WB, HID), _f32),
        pltpu.VMEM_SHARED((NPAD, HID), _f32),
    ],
)
def _sc_deg(dst_hbm, out_hbm, didx, ones_b, buf, acc):
    c = lax.axis_index("c")
    s = lax.axis_index("s")
    wid = c * NS + s
    pltpu.sync_copy(dst_hbm.at[wid], didx)

    _zero_fill(buf, WB)
    one = jnp.ones((L,), _f32)

    def fill_ones(i, carry):
        ones_b[i, :] = one
        return carry

    lax.fori_loop(0, CH, fill_ones, 0)

    for k in range(NWB):
        pltpu.sync_copy(buf, acc.at[pl.ds(s * NPS + k * WB, WB)])
    plsc.subcore_barrier()

    def body(j, carry):
        pltpu.sync_copy(ones_b, acc.at[didx.at[j]], add=True)
        return carry

    lax.fori_loop(0, NCH, body, 0)
    plsc.subcore_barrier()

    for k in range(NWB):
        r0 = s * NPS + k * WB
        pltpu.sync_copy(acc.at[pl.ds(r0, WB)], buf)
        pltpu.sync_copy(buf, out_hbm.at[c].at[pl.ds(r0, WB)])


@functools.partial(
    pl.kernel,
    out_type=jax.ShapeDtypeStruct((NC, NPAD, HID), _f32),
    mesh=_mesh,
    compiler_params=pltpu.CompilerParams(use_tc_tiling_on_sc=False),
    scratch_types=[
        pltpu.VMEM((NCH, CH), jnp.int32),
        pltpu.VMEM((NCH, CH), jnp.int32),
        pltpu.VMEM((CH, HID), _f32),
        pltpu.VMEM((CH, HID), _f32),
        pltpu.VMEM((NPS, HID), _f32),
        pltpu.VMEM((WB, HID), _f32),
        pltpu.VMEM_SHARED((NPAD, HID), _f32),
        pltpu.VMEM_SHARED((NPAD, HID), _f32),
        pltpu.SemaphoreType.DMA,
        pltpu.SemaphoreType.DMA,
    ],
)
def _sc_agg(g_hbm, src_hbm, dst_hbm, out_hbm, sidx, didx, ra, rb, stg, buf,
            gs, acc, sa, sb):
    c = lax.axis_index("c")
    s = lax.axis_index("s")
    wid = c * NS + s
    pltpu.sync_copy(src_hbm.at[wid], sidx)
    pltpu.sync_copy(dst_hbm.at[wid], didx)

    t0 = s * NPS
    pltpu.sync_copy(g_hbm.at[pl.ds(t0, NPS)], stg)
    pltpu.sync_copy(stg, gs.at[pl.ds(t0, NPS)])

    _zero_fill(buf, WB)
    for k in range(NWB):
        pltpu.sync_copy(buf, acc.at[pl.ds(s * NPS + k * WB, WB)])
    plsc.subcore_barrier()

    def start(j, r, sem):
        pltpu.async_copy(gs.at[sidx.at[j]], r, sem)

    def wait(j, r, sem):
        pltpu.make_async_copy(gs.at[sidx.at[j]], r, sem).wait()

    def scat(j, r):
        pltpu.sync_copy(r, acc.at[didx.at[j]], add=True)

    start(0, ra, sa)

    def body(p, carry):
        j0 = 2 * p
        j1 = j0 + 1
        start(j1, rb, sb)
        wait(j0, ra, sa)
        scat(j0, ra)
        start(j0 + 2, ra, sa)
        wait(j1, rb, sb)
        scat(j1, rb)
        return carry

    lax.fori_loop(0, (NCH - 1) // 2, body, 0)
    wait(NCH - 1, ra, sa)
    scat(NCH - 1, ra)
    plsc.subcore_barrier()

    for k in range(NWB):
        r0 = s * NPS + k * WB
        pltpu.sync_copy(acc.at[pl.ds(r0, WB)], buf)
        pltpu.sync_copy(buf, out_hbm.at[c].at[pl.ds(r0, WB)])


RF = NPAD * HID // 128
RN = N * HID // 128


def _tc_h1_body(x2_ref, w1b_ref, h_ref):
    h_ref[...] = jnp.dot(x2_ref[...], w1b_ref[...], preferred_element_type=_f32)


def _tc_g1_body(h_ref, degp_ref, dis_ref, g1_ref):
    dis = lax.rsqrt(degp_ref[0] + degp_ref[1] + 1.0)
    dis_ref[...] = dis
    g1_ref[:RN] = h_ref[...] * dis[:RN]
    g1_ref[RN:] = jnp.zeros((RF - RN, 128), _f32)


def _tc_g2_body(aggp_ref, g1_ref, dis_ref, b1_ref, g2_ref):
    dis = dis_ref[...]
    t = dis * (aggp_ref[0] + aggp_ref[1] + g1_ref[...]) + b1_ref[...]
    g2_ref[...] = jnp.maximum(t, 0.0) * dis


def _tc_out_body(aggp_ref, g2_ref, dis_ref, w2b_ref, b2_ref, o_ref):
    t = dis_ref[:RN] * (aggp_ref[0, :RN] + aggp_ref[1, :RN] + g2_ref[:RN])
    o_ref[...] = jnp.dot(t, w2b_ref[...], preferred_element_type=_f32) + b2_ref[...]


def kernel(x, edge_index, W1, b1, W2, b2):
    src3 = edge_index[0].reshape(NW, NCH, CH)
    dst3 = edge_index[1].reshape(NW, NCH, CH)

    x2 = x.reshape(RN, F_IN * N // RN)
    w1b = jnp.kron(jnp.eye(8, dtype=_f32), W1)
    w2b = jnp.kron(jnp.eye(8, dtype=_f32), W2)
    b1b = jnp.tile(b1, 8).reshape(1, 128)
    b2b = jnp.tile(b2, 8).reshape(1, 8 * CLS)

    h_f = pl.pallas_call(
        _tc_h1_body,
        out_shape=jax.ShapeDtypeStruct((RN, 128), _f32),
    )(x2, w1b)

    degp = _sc_deg(dst3)

    dis_f, g1_f = pl.pallas_call(
        _tc_g1_body,
        out_shape=[
            jax.ShapeDtypeStruct((RF, 128), _f32),
            jax.ShapeDtypeStruct((RF, 128), _f32),
        ],
    )(h_f, degp.reshape(NC, RF, 128))

    aggp1 = _sc_agg(g1_f.reshape(NPAD, HID), src3, dst3)

    g2_f = pl.pallas_call(
        _tc_g2_body,
        out_shape=jax.ShapeDtypeStruct((RF, 128), _f32),
    )(aggp1.reshape(NC, RF, 128), g1_f, dis_f, b1b)

    aggp2 = _sc_agg(g2_f.reshape(NPAD, HID), src3, dst3)

    out_f = pl.pallas_call(
        _tc_out_body,
        out_shape=jax.ShapeDtypeStruct((RN, 8 * CLS), _f32),
    )(aggp2.reshape(NC, RF, 128), g2_f, dis_f, w2b, b2b)

    return out_f.reshape(N, CLS)


# speedup vs baseline: 78.8817x; 1.0132x over previous
import functools

import jax
import jax.numpy as jnp
from jax import lax
from jax.experimental import pallas as pl
from jax.experimental.pallas import tpu as pltpu
from jax.experimental.pallas import tpu_sc as plsc

N = 10000
E = 320000
F_IN = 128
HID = 16
CLS = 40

NC = 2
NS = 16
L = 16

NW = NC * NS
EPW = E // NW
CH = 80
NCH = EPW // CH
NPAD = 10240
NPS = NPAD // NS
WB = 128
NWB = NPS // WB

_mesh = plsc.VectorSubcoreMesh(core_axis_name="c", subcore_axis_name="s")

_f32 = jnp.float32


def _zero_fill(buf, nrows):
    zero = jnp.zeros((L,), _f32)

    def fill(i, carry):
        buf[i, :] = zero
        return carry

    lax.fori_loop(0, nrows, fill, 0)


@functools.partial(
    pl.kernel,
    out_type=jax.ShapeDtypeStruct((NC, NPAD, HID), _f32),
    mesh=_mesh,
    compiler_params=pltpu.CompilerParams(use_tc_tiling_on_sc=False),
    scratch_types=[
        pltpu.VMEM((NCH, CH), jnp.int32),
        pltpu.VMEM((CH, HID), _f32),
        pltpu.VMEM((WB, HID), _f32),
        pltpu.VMEM_SHARED((NPAD, HID), _f32),
        pltpu.SemaphoreType.DMA,
    ],
)
def _sc_deg(dst_hbm, out_hbm, didx, ones_b, buf, acc, sem):
    c = lax.axis_index("c")
    s = lax.axis_index("s")
    wid = c * NS + s
    pltpu.sync_copy(dst_hbm.at[wid], didx)

    _zero_fill(buf, WB)
    one = jnp.ones((L,), _f32)

    def fill_ones(i, carry):
        ones_b[i, :] = one
        return carry

    lax.fori_loop(0, CH, fill_ones, 0)

    for k in range(NWB):
        pltpu.sync_copy(buf, acc.at[pl.ds(s * NPS + k * WB, WB)])
    plsc.subcore_barrier()

    DW = 4

    def body(j, carry):
        pltpu.async_copy(ones_b, acc.at[didx.at[j]], sem, add=True)

        @pl.when(j >= DW)
        def _():
            pltpu.make_async_copy(ones_b, acc.at[didx.at[j - DW]], sem).wait()

        return carry

    lax.fori_loop(0, NCH, body, 0)
    for k in range(DW):
        pltpu.make_async_copy(ones_b, acc.at[didx.at[NCH - DW + k]], sem).wait()
    plsc.subcore_barrier()

    for k in range(NWB):
        r0 = s * NPS + k * WB
        pltpu.sync_copy(acc.at[pl.ds(r0, WB)], buf)
        pltpu.sync_copy(buf, out_hbm.at[c].at[pl.ds(r0, WB)])


@functools.partial(
    pl.kernel,
    out_type=jax.ShapeDtypeStruct((NC, NPAD, HID), _f32),
    mesh=_mesh,
    compiler_params=pltpu.CompilerParams(use_tc_tiling_on_sc=False),
    scratch_types=[
        pltpu.VMEM((NCH, CH), jnp.int32),
        pltpu.VMEM((NCH, CH), jnp.int32),
        pltpu.VMEM((4, CH, HID), _f32),
        pltpu.VMEM((NPS, HID), _f32),
        pltpu.VMEM((WB, HID), _f32),
        pltpu.VMEM_SHARED((NPAD, HID), _f32),
        pltpu.VMEM_SHARED((NPAD, HID), _f32),
        pltpu.SemaphoreType.DMA((4,)),
        pltpu.SemaphoreType.DMA((4,)),
    ],
)
def _sc_agg(g_hbm, src_hbm, dst_hbm, out_hbm, sidx, didx, rows, stg, buf,
            gs, acc, gsem, ssem):
    c = lax.axis_index("c")
    s = lax.axis_index("s")
    wid = c * NS + s
    pltpu.sync_copy(src_hbm.at[wid], sidx)
    pltpu.sync_copy(dst_hbm.at[wid], didx)

    t0 = s * NPS
    pltpu.sync_copy(g_hbm.at[pl.ds(t0, NPS)], stg)
    pltpu.sync_copy(stg, gs.at[pl.ds(t0, NPS)])

    _zero_fill(buf, WB)
    for k in range(NWB):
        pltpu.sync_copy(buf, acc.at[pl.ds(s * NPS + k * WB, WB)])
    plsc.subcore_barrier()

    def start_g(j, b):
        pltpu.async_copy(gs.at[sidx.at[j]], rows.at[b], gsem.at[b])

    def wait_g(j, b):
        pltpu.make_async_copy(gs.at[sidx.at[j]], rows.at[b], gsem.at[b]).wait()

    def start_s(j, b):
        pltpu.async_copy(rows.at[b], acc.at[didx.at[j]], ssem.at[b], add=True)

    def wait_s(j, b):
        pltpu.make_async_copy(rows.at[b], acc.at[didx.at[j]], ssem.at[b]).wait()

    NG = (NCH - 1) // 4
    for b in range(4):
        start_g(b, b)

    def body(q, carry):
        for b in range(4):
            j = 4 * q + b
            wait_g(j, b)
            start_s(j, b)
        for b in range(4):
            j = 4 * q + b
            wait_s(j, b)
            start_g(j + 4, b)
        return carry

    lax.fori_loop(0, NG - 1, body, 0)
    for b in range(4):
        j = 4 * (NG - 1) + b
        wait_g(j, b)
        start_s(j, b)
    wait_s(4 * (NG - 1), 0)
    start_g(NCH - 1, 0)
    wait_g(NCH - 1, 0)
    start_s(NCH - 1, 0)
    for b in range(1, 4):
        wait_s(4 * (NG - 1) + b, b)
    wait_s(NCH - 1, 0)
    plsc.subcore_barrier()

    for k in range(NWB):
        r0 = s * NPS + k * WB
        pltpu.sync_copy(acc.at[pl.ds(r0, WB)], buf)
        pltpu.sync_copy(buf, out_hbm.at[c].at[pl.ds(r0, WB)])


RF = NPAD * HID // 128
RN = N * HID // 128


def _tc_h1_body(x2_ref, w1b_ref, h_ref):
    h_ref[...] = jnp.dot(x2_ref[...], w1b_ref[...], preferred_element_type=_f32)


def _tc_g1_body(h_ref, degp_ref, dis_ref, g1_ref):
    dis = lax.rsqrt(degp_ref[0] + degp_ref[1] + 1.0)
    dis_ref[...] = dis
    g1_ref[:RN] = h_ref[...] * dis[:RN]
    g1_ref[RN:] = jnp.zeros((RF - RN, 128), _f32)


def _tc_g2_body(aggp_ref, g1_ref, dis_ref, b1_ref, g2_ref):
    dis = dis_ref[...]
    t = dis * (aggp_ref[0] + aggp_ref[1] + g1_ref[...]) + b1_ref[...]
    g2_ref[...] = jnp.maximum(t, 0.0) * dis


def _tc_out_body(aggp_ref, g2_ref, dis_ref, w2b_ref, b2_ref, o_ref):
    t = dis_ref[:RN] * (aggp_ref[0, :RN] + aggp_ref[1, :RN] + g2_ref[:RN])
    o_ref[...] = jnp.dot(t, w2b_ref[...], preferred_element_type=_f32) + b2_ref[...]


def kernel(x, edge_index, W1, b1, W2, b2):
    dst3 = edge_index[1].reshape(NW, NCH, CH)
    src3 = lax.optimization_barrier(edge_index)[0].reshape(NW, NCH, CH)

    x2 = x.reshape(RN, F_IN * N // RN)
    w1b = jnp.kron(jnp.eye(8, dtype=_f32), W1)
    w2b = jnp.kron(jnp.eye(8, dtype=_f32), W2)
    b1b = jnp.tile(b1, 8).reshape(1, 128)
    b2b = jnp.tile(b2, 8).reshape(1, 8 * CLS)

    h_f = pl.pallas_call(
        _tc_h1_body,
        out_shape=jax.ShapeDtypeStruct((RN, 128), _f32),
    )(x2, w1b)

    degp = _sc_deg(dst3)

    dis_f, g1_f = pl.pallas_call(
        _tc_g1_body,
        out_shape=[
            jax.ShapeDtypeStruct((RF, 128), _f32),
            jax.ShapeDtypeStruct((RF, 128), _f32),
        ],
    )(h_f, degp.reshape(NC, RF, 128))

    aggp1 = _sc_agg(g1_f.reshape(NPAD, HID), src3, dst3)

    g2_f = pl.pallas_call(
        _tc_g2_body,
        out_shape=jax.ShapeDtypeStruct((RF, 128), _f32),
    )(aggp1.reshape(NC, RF, 128), g1_f, dis_f, b1b)

    aggp2 = _sc_agg(g2_f.reshape(NPAD, HID), src3, dst3)

    out_f = pl.pallas_call(
        _tc_out_body,
        out_shape=jax.ShapeDtypeStruct((RN, 8 * CLS), _f32),
    )(aggp2.reshape(NC, RF, 128), g2_f, dis_f, w2b, b2b)

    return out_f.reshape(N, CLS)


# speedup vs baseline: 91.5771x; 1.1609x over previous
import functools

import jax
import jax.numpy as jnp
from jax import lax
from jax.experimental import pallas as pl
from jax.experimental.pallas import tpu as pltpu
from jax.experimental.pallas import tpu_sc as plsc

N = 10000
E = 320000
F_IN = 128
HID = 16
CLS = 40

NC = 2
NS = 16
L = 16

NW = NC * NS
CH = 128
NCHT = E // CH
CPW = NCHT // NW
XTRA = NCHT - NW * CPW
NPAD = 10240
NPS = NPAD // NS
WB = 128
NWB = NPS // WB

_mesh = plsc.VectorSubcoreMesh(core_axis_name="c", subcore_axis_name="s")

_f32 = jnp.float32


def _zero_fill(buf, nrows):
    zero = jnp.zeros((L,), _f32)

    def fill(i, carry):
        buf[i, :] = zero
        return carry

    lax.fori_loop(0, nrows, fill, 0)


def _load_chunks(ei_hbm, row, tbase, extra, idx):
    pltpu.sync_copy(ei_hbm.at[row].at[pl.ds(tbase, CPW)], idx.at[pl.ds(0, CPW)])

    @pl.when(extra)
    def _():
        pltpu.sync_copy(ei_hbm.at[row].at[pl.ds(tbase + CPW, 1)],
                        idx.at[pl.ds(CPW, 1)])


@functools.partial(
    pl.kernel,
    out_type=jax.ShapeDtypeStruct((NC, NPAD, HID), _f32),
    mesh=_mesh,
    compiler_params=pltpu.CompilerParams(use_tc_tiling_on_sc=False),
    scratch_types=[
        pltpu.VMEM((CPW + 1, CH), jnp.int32),
        pltpu.VMEM((CH, HID), _f32),
        pltpu.VMEM((WB, HID), _f32),
        pltpu.VMEM_SHARED((NPAD, HID), _f32),
        pltpu.SemaphoreType.DMA,
    ],
)
def _sc_deg(ei_hbm, out_hbm, didx, ones_b, buf, acc, sem):
    c = lax.axis_index("c")
    s = lax.axis_index("s")
    wid = c * NS + s
    tbase = wid * CPW + jnp.minimum(wid, XTRA)
    extra = wid < XTRA
    _load_chunks(ei_hbm, 1, tbase, extra, didx)

    _zero_fill(buf, WB)
    one = jnp.ones((L,), _f32)

    def fill_ones(i, carry):
        ones_b[i, :] = one
        return carry

    lax.fori_loop(0, CH, fill_ones, 0)

    for k in range(NWB):
        pltpu.sync_copy(buf, acc.at[pl.ds(s * NPS + k * WB, WB)])
    plsc.subcore_barrier()

    DW = 4

    def body(j, carry):
        pltpu.async_copy(ones_b, acc.at[didx.at[j]], sem, add=True)

        @pl.when(j >= DW)
        def _():
            pltpu.make_async_copy(ones_b, acc.at[didx.at[j - DW]], sem).wait()

        return carry

    lax.fori_loop(0, CPW, body, 0)
    for k in range(DW):
        pltpu.make_async_copy(ones_b, acc.at[didx.at[CPW - DW + k]], sem).wait()

    @pl.when(extra)
    def _():
        pltpu.sync_copy(ones_b, acc.at[didx.at[CPW]], add=True)

    plsc.subcore_barrier()

    for k in range(NWB):
        r0 = s * NPS + k * WB
        pltpu.sync_copy(acc.at[pl.ds(r0, WB)], buf)
        pltpu.sync_copy(buf, out_hbm.at[c].at[pl.ds(r0, WB)])


@functools.partial(
    pl.kernel,
    out_type=jax.ShapeDtypeStruct((NC, NPAD, HID), _f32),
    mesh=_mesh,
    compiler_params=pltpu.CompilerParams(use_tc_tiling_on_sc=False),
    scratch_types=[
        pltpu.VMEM((CPW + 1, CH), jnp.int32),
        pltpu.VMEM((CPW + 1, CH), jnp.int32),
        pltpu.VMEM((4, CH, HID), _f32),
        pltpu.VMEM((NPS, HID), _f32),
        pltpu.VMEM((WB, HID), _f32),
        pltpu.VMEM_SHARED((NPAD, HID), _f32),
        pltpu.VMEM_SHARED((NPAD, HID), _f32),
        pltpu.SemaphoreType.DMA((4,)),
        pltpu.SemaphoreType.DMA((4,)),
    ],
)
def _sc_agg(g_hbm, ei_hbm, out_hbm, sidx, didx, rows, stg, buf,
            gs, acc, gsem, ssem):
    c = lax.axis_index("c")
    s = lax.axis_index("s")
    wid = c * NS + s
    tbase = wid * CPW + jnp.minimum(wid, XTRA)
    extra = wid < XTRA
    _load_chunks(ei_hbm, 0, tbase, extra, sidx)
    _load_chunks(ei_hbm, 1, tbase, extra, didx)

    t0 = s * NPS
    pltpu.sync_copy(g_hbm.at[pl.ds(t0, NPS)], stg)
    pltpu.sync_copy(stg, gs.at[pl.ds(t0, NPS)])

    _zero_fill(buf, WB)
    for k in range(NWB):
        pltpu.sync_copy(buf, acc.at[pl.ds(s * NPS + k * WB, WB)])
    plsc.subcore_barrier()

    def start_g(j, b):
        pltpu.async_copy(gs.at[sidx.at[j]], rows.at[b], gsem.at[b])

    def wait_g(j, b):
        pltpu.make_async_copy(gs.at[sidx.at[j]], rows.at[b], gsem.at[b]).wait()

    def start_s(j, b):
        pltpu.async_copy(rows.at[b], acc.at[didx.at[j]], ssem.at[b], add=True)

    def wait_s(j, b):
        pltpu.make_async_copy(rows.at[b], acc.at[didx.at[j]], ssem.at[b]).wait()

    NG = CPW // 4
    for b in range(4):
        start_g(b, b)

    def body(q, carry):
        for b in range(4):
            j = 4 * q + b
            wait_g(j, b)
            start_s(j, b)
        for b in range(4):
            j = 4 * q + b
            wait_s(j, b)
            start_g(j + 4, b)
        return carry

    lax.fori_loop(0, NG - 1, body, 0)
    for b in range(4):
        j = 4 * (NG - 1) + b
        wait_g(j, b)
        start_s(j, b)
    j0 = 4 * NG
    wait_s(j0 - 4, 0)
    start_g(j0, 0)
    wait_s(j0 - 3, 1)
    start_g(j0 + 1, 1)
    wait_s(j0 - 2, 2)

    @pl.when(extra)
    def _():
        start_g(j0 + 2, 2)

    wait_s(j0 - 1, 3)
    wait_g(j0, 0)
    start_s(j0, 0)
    wait_g(j0 + 1, 1)
    start_s(j0 + 1, 1)

    @pl.when(extra)
    def _():
        wait_g(j0 + 2, 2)
        start_s(j0 + 2, 2)
        wait_s(j0 + 2, 2)

    wait_s(j0, 0)
    wait_s(j0 + 1, 1)
    plsc.subcore_barrier()

    for k in range(NWB):
        r0 = s * NPS + k * WB
        pltpu.sync_copy(acc.at[pl.ds(r0, WB)], buf)
        pltpu.sync_copy(buf, out_hbm.at[c].at[pl.ds(r0, WB)])


RF = NPAD * HID // 128
RN = N * HID // 128


def _tc_h1_body(x2_ref, w1b_ref, h_ref):
    h_ref[...] = jnp.dot(x2_ref[...], w1b_ref[...], preferred_element_type=_f32)


def _tc_g1_body(h_ref, degp_ref, dis_ref, g1_ref):
    dis = lax.rsqrt(degp_ref[0] + degp_ref[1] + 1.0)
    dis_ref[...] = dis
    g1_ref[:RN] = h_ref[...] * dis[:RN]
    g1_ref[RN:] = jnp.zeros((RF - RN, 128), _f32)


def _tc_g2_body(aggp_ref, g1_ref, dis_ref, b1_ref, g2_ref):
    dis = dis_ref[...]
    t = dis * (aggp_ref[0] + aggp_ref[1] + g1_ref[...]) + b1_ref[...]
    g2_ref[...] = jnp.maximum(t, 0.0) * dis


def _tc_out_body(aggp_ref, g2_ref, dis_ref, w2b_ref, b2_ref, o_ref):
    t = dis_ref[:RN] * (aggp_ref[0, :RN] + aggp_ref[1, :RN] + g2_ref[:RN])
    o_ref[...] = jnp.dot(t, w2b_ref[...], preferred_element_type=_f32) + b2_ref[...]


def kernel(x, edge_index, W1, b1, W2, b2):
    ei2 = edge_index.reshape(2, NCHT, CH)

    x2 = x.reshape(RN, F_IN * N // RN)
    w1b = jnp.kron(jnp.eye(8, dtype=_f32), W1)
    w2b = jnp.kron(jnp.eye(8, dtype=_f32), W2)
    b1b = jnp.tile(b1, 8).reshape(1, 128)
    b2b = jnp.tile(b2, 8).reshape(1, 8 * CLS)

    h_f = pl.pallas_call(
        _tc_h1_body,
        out_shape=jax.ShapeDtypeStruct((RN, 128), _f32),
    )(x2, w1b)

    degp = _sc_deg(ei2)

    dis_f, g1_f = pl.pallas_call(
        _tc_g1_body,
        out_shape=[
            jax.ShapeDtypeStruct((RF, 128), _f32),
            jax.ShapeDtypeStruct((RF, 128), _f32),
        ],
    )(h_f, degp.reshape(NC, RF, 128))

    aggp1 = _sc_agg(g1_f.reshape(NPAD, HID), ei2)

    g2_f = pl.pallas_call(
        _tc_g2_body,
        out_shape=jax.ShapeDtypeStruct((RF, 128), _f32),
    )(aggp1.reshape(NC, RF, 128), g1_f, dis_f, b1b)

    aggp2 = _sc_agg(g2_f.reshape(NPAD, HID), ei2)

    out_f = pl.pallas_call(
        _tc_out_body,
        out_shape=jax.ShapeDtypeStruct((RN, 8 * CLS), _f32),
    )(aggp2.reshape(NC, RF, 128), g2_f, dis_f, w2b, b2b)

    return out_f.reshape(N, CLS)


# speedup vs baseline: 91.8736x; 1.0032x over previous
import functools

import jax
import jax.numpy as jnp
from jax import lax
from jax.experimental import pallas as pl
from jax.experimental.pallas import tpu as pltpu
from jax.experimental.pallas import tpu_sc as plsc

N = 10000
E = 320000
F_IN = 128
HID = 16
CLS = 40

NC = 2
NS = 16
L = 16

NW = NC * NS
CH = 256
NCHT = E // CH
CPW = NCHT // NW
XTRA = NCHT - NW * CPW
NPAD = 10240
NPS = NPAD // NS
WB = 128
NWB = NPS // WB

_mesh = plsc.VectorSubcoreMesh(core_axis_name="c", subcore_axis_name="s")

_f32 = jnp.float32


def _zero_fill(buf, nrows):
    zero = jnp.zeros((L,), _f32)

    def fill(i, carry):
        buf[i, :] = zero
        return carry

    lax.fori_loop(0, nrows, fill, 0)


def _load_chunks(ei_hbm, row, tbase, extra, idx):
    pltpu.sync_copy(ei_hbm.at[row].at[pl.ds(tbase, CPW)], idx.at[pl.ds(0, CPW)])

    @pl.when(extra)
    def _():
        pltpu.sync_copy(ei_hbm.at[row].at[pl.ds(tbase + CPW, 1)],
                        idx.at[pl.ds(CPW, 1)])


@functools.partial(
    pl.kernel,
    out_type=jax.ShapeDtypeStruct((NC, NPAD, HID), _f32),
    mesh=_mesh,
    compiler_params=pltpu.CompilerParams(use_tc_tiling_on_sc=False),
    scratch_types=[
        pltpu.VMEM((CPW + 1, CH), jnp.int32),
        pltpu.VMEM((CH, HID), _f32),
        pltpu.VMEM((WB, HID), _f32),
        pltpu.VMEM_SHARED((NPAD, HID), _f32),
        pltpu.SemaphoreType.DMA,
    ],
)
def _sc_deg(ei_hbm, out_hbm, didx, ones_b, buf, acc, sem):
    c = lax.axis_index("c")
    s = lax.axis_index("s")
    wid = c * NS + s
    tbase = wid * CPW + jnp.minimum(wid, XTRA)
    extra = wid < XTRA
    _load_chunks(ei_hbm, 1, tbase, extra, didx)

    _zero_fill(buf, WB)
    one = jnp.ones((L,), _f32)

    def fill_ones(i, carry):
        ones_b[i, :] = one
        return carry

    lax.fori_loop(0, CH, fill_ones, 0)

    for k in range(NWB):
        pltpu.sync_copy(buf, acc.at[pl.ds(s * NPS + k * WB, WB)])
    plsc.subcore_barrier()

    DW = 4

    def body(j, carry):
        pltpu.async_copy(ones_b, acc.at[didx.at[j]], sem, add=True)

        @pl.when(j >= DW)
        def _():
            pltpu.make_async_copy(ones_b, acc.at[didx.at[j - DW]], sem).wait()

        return carry

    lax.fori_loop(0, CPW, body, 0)
    for k in range(DW):
        pltpu.make_async_copy(ones_b, acc.at[didx.at[CPW - DW + k]], sem).wait()

    @pl.when(extra)
    def _():
        pltpu.sync_copy(ones_b, acc.at[didx.at[CPW]], add=True)

    plsc.subcore_barrier()

    for k in range(NWB):
        r0 = s * NPS + k * WB
        pltpu.sync_copy(acc.at[pl.ds(r0, WB)], buf)
        pltpu.sync_copy(buf, out_hbm.at[c].at[pl.ds(r0, WB)])


@functools.partial(
    pl.kernel,
    out_type=jax.ShapeDtypeStruct((NC, NPAD, HID), _f32),
    mesh=_mesh,
    compiler_params=pltpu.CompilerParams(use_tc_tiling_on_sc=False),
    scratch_types=[
        pltpu.VMEM((CPW + 1, CH), jnp.int32),
        pltpu.VMEM((CPW + 1, CH), jnp.int32),
        pltpu.VMEM((4, CH, HID), _f32),
        pltpu.VMEM((NPS, HID), _f32),
        pltpu.VMEM((WB, HID), _f32),
        pltpu.VMEM_SHARED((NPAD, HID), _f32),
        pltpu.VMEM_SHARED((NPAD, HID), _f32),
        pltpu.SemaphoreType.DMA((4,)),
        pltpu.SemaphoreType.DMA((4,)),
    ],
)
def _sc_agg(g_hbm, ei_hbm, out_hbm, sidx, didx, rows, stg, buf,
            gs, acc, gsem, ssem):
    c = lax.axis_index("c")
    s = lax.axis_index("s")
    wid = c * NS + s
    tbase = wid * CPW + jnp.minimum(wid, XTRA)
    extra = wid < XTRA
    _load_chunks(ei_hbm, 0, tbase, extra, sidx)
    _load_chunks(ei_hbm, 1, tbase, extra, didx)

    t0 = s * NPS
    pltpu.sync_copy(g_hbm.at[pl.ds(t0, NPS)], stg)
    pltpu.sync_copy(stg, gs.at[pl.ds(t0, NPS)])

    _zero_fill(buf, WB)
    for k in range(NWB):
        pltpu.sync_copy(buf, acc.at[pl.ds(s * NPS + k * WB, WB)])
    plsc.subcore_barrier()

    def start_g(j, b):
        pltpu.async_copy(gs.at[sidx.at[j]], rows.at[b], gsem.at[b])

    def wait_g(j, b):
        pltpu.make_async_copy(gs.at[sidx.at[j]], rows.at[b], gsem.at[b]).wait()

    def start_s(j, b):
        pltpu.async_copy(rows.at[b], acc.at[didx.at[j]], ssem.at[b], add=True)

    def wait_s(j, b):
        pltpu.make_async_copy(rows.at[b], acc.at[didx.at[j]], ssem.at[b]).wait()

    NG = CPW // 4
    TAIL = CPW - 4 * NG
    assert 4 * NG >= 4 and TAIL + 1 <= 4
    for b in range(4):
        start_g(b, b)

    def body(q, carry):
        for b in range(4):
            j = 4 * q + b
            wait_g(j, b)
            start_s(j, b)
        for b in range(4):
            j = 4 * q + b
            wait_s(j, b)
            start_g(j + 4, b)
        return carry

    lax.fori_loop(0, NG - 1, body, 0)
    for b in range(4):
        j = 4 * (NG - 1) + b
        wait_g(j, b)
        start_s(j, b)
    j0 = 4 * NG
    for t in range(TAIL):
        wait_s(j0 - 4 + t, t)
        start_g(j0 + t, t)
    wait_s(j0 - 4 + TAIL, TAIL)

    @pl.when(extra)
    def _():
        start_g(CPW, TAIL)

    for b in range(TAIL + 1, 4):
        wait_s(j0 - 4 + b, b)
    for t in range(TAIL):
        wait_g(j0 + t, t)
        start_s(j0 + t, t)

    @pl.when(extra)
    def _():
        wait_g(CPW, TAIL)
        start_s(CPW, TAIL)
        wait_s(CPW, TAIL)

    for t in range(TAIL):
        wait_s(j0 + t, t)
    plsc.subcore_barrier()

    for k in range(NWB):
        r0 = s * NPS + k * WB
        pltpu.sync_copy(acc.at[pl.ds(r0, WB)], buf)
        pltpu.sync_copy(buf, out_hbm.at[c].at[pl.ds(r0, WB)])


RF = NPAD * HID // 128
RN = N * HID // 128


def _tc_h1_body(x2_ref, w1b_ref, h_ref):
    h_ref[...] = jnp.dot(x2_ref[...], w1b_ref[...], preferred_element_type=_f32)


def _tc_g1_body(h_ref, degp_ref, dis_ref, g1_ref):
    dis = lax.rsqrt(degp_ref[0] + degp_ref[1] + 1.0)
    dis_ref[...] = dis
    g1_ref[:RN] = h_ref[...] * dis[:RN]
    g1_ref[RN:] = jnp.zeros((RF - RN, 128), _f32)


def _tc_g2_body(aggp_ref, g1_ref, dis_ref, b1_ref, g2_ref):
    dis = dis_ref[...]
    t = dis * (aggp_ref[0] + aggp_ref[1] + g1_ref[...]) + b1_ref[...]
    g2_ref[...] = jnp.maximum(t, 0.0) * dis


def _tc_out_body(aggp_ref, g2_ref, dis_ref, w2b_ref, b2_ref, o_ref):
    t = dis_ref[:RN] * (aggp_ref[0, :RN] + aggp_ref[1, :RN] + g2_ref[:RN])
    o_ref[...] = jnp.dot(t, w2b_ref[...], preferred_element_type=_f32) + b2_ref[...]


def kernel(x, edge_index, W1, b1, W2, b2):
    ei2 = edge_index.reshape(2, NCHT, CH)

    x2 = x.reshape(RN, F_IN * N // RN)
    w1b = jnp.kron(jnp.eye(8, dtype=_f32), W1)
    w2b = jnp.kron(jnp.eye(8, dtype=_f32), W2)
    b1b = jnp.tile(b1, 8).reshape(1, 128)
    b2b = jnp.tile(b2, 8).reshape(1, 8 * CLS)

    h_f = pl.pallas_call(
        _tc_h1_body,
        out_shape=jax.ShapeDtypeStruct((RN, 128), _f32),
    )(x2, w1b)

    degp = _sc_deg(ei2)

    dis_f, g1_f = pl.pallas_call(
        _tc_g1_body,
        out_shape=[
            jax.ShapeDtypeStruct((RF, 128), _f32),
            jax.ShapeDtypeStruct((RF, 128), _f32),
        ],
    )(h_f, degp.reshape(NC, RF, 128))

    aggp1 = _sc_agg(g1_f.reshape(NPAD, HID), ei2)

    g2_f = pl.pallas_call(
        _tc_g2_body,
        out_shape=jax.ShapeDtypeStruct((RF, 128), _f32),
    )(aggp1.reshape(NC, RF, 128), g1_f, dis_f, b1b)

    aggp2 = _sc_agg(g2_f.reshape(NPAD, HID), ei2)

    out_f = pl.pallas_call(
        _tc_out_body,
        out_shape=jax.ShapeDtypeStruct((RN, 8 * CLS), _f32),
    )(aggp2.reshape(NC, RF, 128), g2_f, dis_f, w2b, b2b)

    return out_f.reshape(N, CLS)


# speedup vs baseline: 101.4106x; 1.1038x over previous
import functools

import jax
import jax.numpy as jnp
from jax import lax
from jax.experimental import pallas as pl
from jax.experimental.pallas import tpu as pltpu
from jax.experimental.pallas import tpu_sc as plsc

N = 10000
E = 320000
F_IN = 128
HID = 16
CLS = 40

NC = 2
NS = 16
L = 16

NW = NC * NS
CH = 256
NCHT = E // CH
CPW = NCHT // NW
XTRA = NCHT - NW * CPW
NPAD = 10240
NPS = NPAD // NS
WB = 128
NWB = NPS // WB

_mesh = plsc.VectorSubcoreMesh(core_axis_name="c", subcore_axis_name="s")

_f32 = jnp.float32


def _zero_fill(buf, nrows):
    zero = jnp.zeros((L,), _f32)

    def fill(i, carry):
        buf[i, :] = zero
        return carry

    lax.fori_loop(0, nrows, fill, 0)


def _load_chunks(ei_hbm, row, tbase, extra, idx):
    pltpu.sync_copy(ei_hbm.at[row].at[pl.ds(tbase, CPW)], idx.at[pl.ds(0, CPW)])

    @pl.when(extra)
    def _():
        pltpu.sync_copy(ei_hbm.at[row].at[pl.ds(tbase + CPW, 1)],
                        idx.at[pl.ds(CPW, 1)])


@functools.partial(
    pl.kernel,
    out_type=jax.ShapeDtypeStruct((NC, NPAD, HID), _f32),
    mesh=_mesh,
    compiler_params=pltpu.CompilerParams(use_tc_tiling_on_sc=False),
    scratch_types=[
        pltpu.VMEM((CPW + 1, CH), jnp.int32),
        pltpu.VMEM((CH, HID), _f32),
        pltpu.VMEM((2, WB, HID), _f32),
        pltpu.VMEM_SHARED((NPAD, HID), _f32),
        pltpu.SemaphoreType.DMA((4,)),
    ],
)
def _sc_deg(ei_hbm, out_hbm, didx, ones_b, bufs, acc, sems):
    c = lax.axis_index("c")
    s = lax.axis_index("s")
    wid = c * NS + s
    tbase = wid * CPW + jnp.minimum(wid, XTRA)
    extra = wid < XTRA
    sem = sems.at[3]
    buf = bufs.at[0]

    cp_di = pltpu.async_copy(ei_hbm.at[1].at[pl.ds(tbase, CPW)],
                             didx.at[pl.ds(0, CPW)], sems.at[2])

    @pl.when(extra)
    def _():
        pltpu.async_copy(ei_hbm.at[1].at[pl.ds(tbase + CPW, 1)],
                         didx.at[pl.ds(CPW, 1)], sems.at[1])

    _zero_fill(buf, WB)
    one = jnp.ones((L,), _f32)

    def fill_ones(i, carry):
        ones_b[i, :] = one
        return carry

    lax.fori_loop(0, CH, fill_ones, 0)

    for k in range(NWB):
        pltpu.async_copy(buf, acc.at[pl.ds(s * NPS + k * WB, WB)], sems.at[0])
    cp_di.wait()

    @pl.when(extra)
    def _():
        pltpu.make_async_copy(ei_hbm.at[1].at[pl.ds(tbase + CPW, 1)],
                              didx.at[pl.ds(CPW, 1)], sems.at[1]).wait()

    for k in range(NWB):
        pltpu.make_async_copy(buf, acc.at[pl.ds(s * NPS + k * WB, WB)],
                              sems.at[0]).wait()
    plsc.subcore_barrier()

    DW = 4

    def body(j, carry):
        pltpu.async_copy(ones_b, acc.at[didx.at[j]], sem, add=True)

        @pl.when(j >= DW)
        def _():
            pltpu.make_async_copy(ones_b, acc.at[didx.at[j - DW]], sem).wait()

        return carry

    lax.fori_loop(0, CPW, body, 0)
    for k in range(DW):
        pltpu.make_async_copy(ones_b, acc.at[didx.at[CPW - DW + k]], sem).wait()

    @pl.when(extra)
    def _():
        pltpu.sync_copy(ones_b, acc.at[didx.at[CPW]], add=True)

    plsc.subcore_barrier()

    for k in range(NWB):
        b = k % 2
        r0 = s * NPS + k * WB
        if k >= 2:
            rp = s * NPS + (k - 2) * WB
            pltpu.make_async_copy(bufs.at[b], out_hbm.at[c].at[pl.ds(rp, WB)],
                                  sems.at[b]).wait()
        pltpu.sync_copy(acc.at[pl.ds(r0, WB)], bufs.at[b])
        pltpu.async_copy(bufs.at[b], out_hbm.at[c].at[pl.ds(r0, WB)], sems.at[b])
    for k in (NWB - 2, NWB - 1):
        b = k % 2
        r0 = s * NPS + k * WB
        pltpu.make_async_copy(bufs.at[b], out_hbm.at[c].at[pl.ds(r0, WB)],
                              sems.at[b]).wait()


@functools.partial(
    pl.kernel,
    out_type=jax.ShapeDtypeStruct((NC, NPAD, HID), _f32),
    mesh=_mesh,
    compiler_params=pltpu.CompilerParams(use_tc_tiling_on_sc=False),
    scratch_types=[
        pltpu.VMEM((CPW + 1, CH), jnp.int32),
        pltpu.VMEM((CPW + 1, CH), jnp.int32),
        pltpu.VMEM((4, CH, HID), _f32),
        pltpu.VMEM((NPS, HID), _f32),
        pltpu.VMEM((WB, HID), _f32),
        pltpu.VMEM_SHARED((NPAD, HID), _f32),
        pltpu.VMEM_SHARED((NPAD, HID), _f32),
        pltpu.SemaphoreType.DMA((4,)),
        pltpu.SemaphoreType.DMA((4,)),
    ],
)
def _sc_agg(g_hbm, ei_hbm, out_hbm, sidx, didx, rows, stg, buf,
            gs, acc, gsem, ssem):
    c = lax.axis_index("c")
    s = lax.axis_index("s")
    wid = c * NS + s
    tbase = wid * CPW + jnp.minimum(wid, XTRA)
    extra = wid < XTRA
    t0 = s * NPS

    cp_si = pltpu.async_copy(ei_hbm.at[0].at[pl.ds(tbase, CPW)],
                             sidx.at[pl.ds(0, CPW)], gsem.at[0])
    cp_di = pltpu.async_copy(ei_hbm.at[1].at[pl.ds(tbase, CPW)],
                             didx.at[pl.ds(0, CPW)], gsem.at[1])
    cp_st = pltpu.async_copy(g_hbm.at[pl.ds(t0, NPS)], stg, ssem.at[0])

    @pl.when(extra)
    def _():
        pltpu.async_copy(ei_hbm.at[0].at[pl.ds(tbase + CPW, 1)],
                         sidx.at[pl.ds(CPW, 1)], gsem.at[2])
        pltpu.async_copy(ei_hbm.at[1].at[pl.ds(tbase + CPW, 1)],
                         didx.at[pl.ds(CPW, 1)], gsem.at[3])

    _zero_fill(buf, WB)
    for k in range(NWB):
        pltpu.async_copy(buf, acc.at[pl.ds(t0 + k * WB, WB)], ssem.at[1])
    cp_st.wait()
    cp_gs = pltpu.async_copy(stg, gs.at[pl.ds(t0, NPS)], ssem.at[2])
    cp_si.wait()
    cp_di.wait()

    @pl.when(extra)
    def _():
        pltpu.make_async_copy(ei_hbm.at[0].at[pl.ds(tbase + CPW, 1)],
                              sidx.at[pl.ds(CPW, 1)], gsem.at[2]).wait()
        pltpu.make_async_copy(ei_hbm.at[1].at[pl.ds(tbase + CPW, 1)],
                              didx.at[pl.ds(CPW, 1)], gsem.at[3]).wait()

    for k in range(NWB):
        pltpu.make_async_copy(buf, acc.at[pl.ds(t0 + k * WB, WB)],
                              ssem.at[1]).wait()
    cp_gs.wait()
    plsc.subcore_barrier()

    def start_g(j, b):
        pltpu.async_copy(gs.at[sidx.at[j]], rows.at[b], gsem.at[b])

    def wait_g(j, b):
        pltpu.make_async_copy(gs.at[sidx.at[j]], rows.at[b], gsem.at[b]).wait()

    def start_s(j, b):
        pltpu.async_copy(rows.at[b], acc.at[didx.at[j]], ssem.at[b], add=True)

    def wait_s(j, b):
        pltpu.make_async_copy(rows.at[b], acc.at[didx.at[j]], ssem.at[b]).wait()

    NG = CPW // 4
    TAIL = CPW - 4 * NG
    assert 4 * NG >= 4 and TAIL + 1 <= 4
    for b in range(4):
        start_g(b, b)

    def body(q, carry):
        for b in range(4):
            j = 4 * q + b
            wait_g(j, b)
            start_s(j, b)
        for b in range(4):
            j = 4 * q + b
            wait_s(j, b)
            start_g(j + 4, b)
        return carry

    lax.fori_loop(0, NG - 1, body, 0)
    for b in range(4):
        j = 4 * (NG - 1) + b
        wait_g(j, b)
        start_s(j, b)
    j0 = 4 * NG
    for t in range(TAIL):
        wait_s(j0 - 4 + t, t)
        start_g(j0 + t, t)
    wait_s(j0 - 4 + TAIL, TAIL)

    @pl.when(extra)
    def _():
        start_g(CPW, TAIL)

    for b in range(TAIL + 1, 4):
        wait_s(j0 - 4 + b, b)
    for t in range(TAIL):
        wait_g(j0 + t, t)
        start_s(j0 + t, t)

    @pl.when(extra)
    def _():
        wait_g(CPW, TAIL)
        start_s(CPW, TAIL)
        wait_s(CPW, TAIL)

    for t in range(TAIL):
        wait_s(j0 + t, t)
    plsc.subcore_barrier()

    def bounce(b):
        return rows.at[b].at[pl.ds(0, WB)]

    for k in range(NWB):
        b = k % 2
        r0 = s * NPS + k * WB
        if k >= 2:
            rp = s * NPS + (k - 2) * WB
            pltpu.make_async_copy(bounce(b), out_hbm.at[c].at[pl.ds(rp, WB)],
                                  gsem.at[b]).wait()
        pltpu.sync_copy(acc.at[pl.ds(r0, WB)], bounce(b))
        pltpu.async_copy(bounce(b), out_hbm.at[c].at[pl.ds(r0, WB)], gsem.at[b])
    for k in (NWB - 2, NWB - 1):
        b = k % 2
        r0 = s * NPS + k * WB
        pltpu.make_async_copy(bounce(b), out_hbm.at[c].at[pl.ds(r0, WB)],
                              gsem.at[b]).wait()


RF = NPAD * HID // 128
RN = N * HID // 128


def _tc_h1_body(x2_ref, w1b_ref, h_ref):
    h_ref[...] = jnp.dot(x2_ref[...], w1b_ref[...], preferred_element_type=_f32)


def _tc_g1_body(h_ref, degp_ref, dis_ref, g1_ref):
    dis = lax.rsqrt(degp_ref[0] + degp_ref[1] + 1.0)
    dis_ref[...] = dis
    g1_ref[:RN] = h_ref[...] * dis[:RN]
    g1_ref[RN:] = jnp.zeros((RF - RN, 128), _f32)


def _tc_g2_body(aggp_ref, g1_ref, dis_ref, b1_ref, g2_ref):
    dis = dis_ref[...]
    t = dis * (aggp_ref[0] + aggp_ref[1] + g1_ref[...]) + b1_ref[...]
    g2_ref[...] = jnp.maximum(t, 0.0) * dis


def _tc_out_body(aggp_ref, g2_ref, dis_ref, w2b_ref, b2_ref, o_ref):
    t = dis_ref[:RN] * (aggp_ref[0, :RN] + aggp_ref[1, :RN] + g2_ref[:RN])
    o_ref[...] = jnp.dot(t, w2b_ref[...], preferred_element_type=_f32) + b2_ref[...]


def kernel(x, edge_index, W1, b1, W2, b2):
    ei2 = edge_index.reshape(2, NCHT, CH)

    x2 = x.reshape(RN, F_IN * N // RN)
    w1b = jnp.kron(jnp.eye(8, dtype=_f32), W1)
    w2b = jnp.kron(jnp.eye(8, dtype=_f32), W2)
    b1b = jnp.tile(b1, 8).reshape(1, 128)
    b2b = jnp.tile(b2, 8).reshape(1, 8 * CLS)

    h_f = pl.pallas_call(
        _tc_h1_body,
        out_shape=jax.ShapeDtypeStruct((RN, 128), _f32),
    )(x2, w1b)

    degp = _sc_deg(ei2)

    dis_f, g1_f = pl.pallas_call(
        _tc_g1_body,
        out_shape=[
            jax.ShapeDtypeStruct((RF, 128), _f32),
            jax.ShapeDtypeStruct((RF, 128), _f32),
        ],
    )(h_f, degp.reshape(NC, RF, 128))

    aggp1 = _sc_agg(g1_f.reshape(NPAD, HID), ei2)

    g2_f = pl.pallas_call(
        _tc_g2_body,
        out_shape=jax.ShapeDtypeStruct((RF, 128), _f32),
    )(aggp1.reshape(NC, RF, 128), g1_f, dis_f, b1b)

    aggp2 = _sc_agg(g2_f.reshape(NPAD, HID), ei2)

    out_f = pl.pallas_call(
        _tc_out_body,
        out_shape=jax.ShapeDtypeStruct((RN, 8 * CLS), _f32),
    )(aggp2.reshape(NC, RF, 128), g2_f, dis_f, w2b, b2b)

    return out_f.reshape(N, CLS)


# speedup vs baseline: 105.2219x; 1.0376x over previous
import functools

import jax
import jax.numpy as jnp
from jax import lax
from jax.experimental import pallas as pl
from jax.experimental.pallas import tpu as pltpu
from jax.experimental.pallas import tpu_sc as plsc

N = 10000
E = 320000
F_IN = 128
HID = 16
CLS = 40

NC = 2
NS = 16
L = 16

NW = NC * NS
CH = 256
NCHT = E // CH
CPW = NCHT // NW
XTRA = NCHT - NW * CPW
NPAD = 10240
NPS = NPAD // NS
RB = 8
WB = 128
NWB = NPS // WB

_mesh = plsc.VectorSubcoreMesh(core_axis_name="c", subcore_axis_name="s")

_f32 = jnp.float32


def _zero_fill(buf, nrows):
    zero = jnp.zeros((L,), _f32)

    def fill(i, carry):
        buf[i, :] = zero
        return carry

    lax.fori_loop(0, nrows, fill, 0)


def _load_chunks(ei_hbm, row, tbase, extra, idx):
    pltpu.sync_copy(ei_hbm.at[row].at[pl.ds(tbase, CPW)], idx.at[pl.ds(0, CPW)])

    @pl.when(extra)
    def _():
        pltpu.sync_copy(ei_hbm.at[row].at[pl.ds(tbase + CPW, 1)],
                        idx.at[pl.ds(CPW, 1)])


@functools.partial(
    pl.kernel,
    out_type=jax.ShapeDtypeStruct((NC, NPAD, HID), _f32),
    mesh=_mesh,
    compiler_params=pltpu.CompilerParams(use_tc_tiling_on_sc=False),
    scratch_types=[
        pltpu.VMEM((CPW + 1, CH), jnp.int32),
        pltpu.VMEM((CH, HID), _f32),
        pltpu.VMEM((2, WB, HID), _f32),
        pltpu.VMEM_SHARED((NPAD, HID), _f32),
        pltpu.SemaphoreType.DMA((4,)),
    ],
)
def _sc_deg(ei_hbm, out_hbm, didx, ones_b, bufs, acc, sems):
    c = lax.axis_index("c")
    s = lax.axis_index("s")
    wid = c * NS + s
    tbase = wid * CPW + jnp.minimum(wid, XTRA)
    extra = wid < XTRA
    sem = sems.at[3]
    buf = bufs.at[0]

    cp_di = pltpu.async_copy(ei_hbm.at[1].at[pl.ds(tbase, CPW)],
                             didx.at[pl.ds(0, CPW)], sems.at[2])

    @pl.when(extra)
    def _():
        pltpu.async_copy(ei_hbm.at[1].at[pl.ds(tbase + CPW, 1)],
                         didx.at[pl.ds(CPW, 1)], sems.at[1])

    _zero_fill(buf, WB)
    one = jnp.ones((L,), _f32)

    def fill_ones(i, carry):
        ones_b[i, :] = one
        return carry

    lax.fori_loop(0, CH, fill_ones, 0)

    for k in range(NWB):
        pltpu.async_copy(buf, acc.at[pl.ds(s * NPS + k * WB, WB)], sems.at[0])
    cp_di.wait()

    @pl.when(extra)
    def _():
        pltpu.make_async_copy(ei_hbm.at[1].at[pl.ds(tbase + CPW, 1)],
                              didx.at[pl.ds(CPW, 1)], sems.at[1]).wait()

    for k in range(NWB):
        pltpu.make_async_copy(buf, acc.at[pl.ds(s * NPS + k * WB, WB)],
                              sems.at[0]).wait()
    plsc.subcore_barrier()

    DW = 8

    def body(j, carry):
        pltpu.async_copy(ones_b, acc.at[didx.at[j]], sem, add=True)

        @pl.when(j >= DW)
        def _():
            pltpu.make_async_copy(ones_b, acc.at[didx.at[j - DW]], sem).wait()

        return carry

    lax.fori_loop(0, CPW, body, 0)
    for k in range(DW):
        pltpu.make_async_copy(ones_b, acc.at[didx.at[CPW - DW + k]], sem).wait()

    @pl.when(extra)
    def _():
        pltpu.sync_copy(ones_b, acc.at[didx.at[CPW]], add=True)

    plsc.subcore_barrier()

    for k in range(NWB):
        b = k % 2
        r0 = s * NPS + k * WB
        if k >= 2:
            rp = s * NPS + (k - 2) * WB
            pltpu.make_async_copy(bufs.at[b], out_hbm.at[c].at[pl.ds(rp, WB)],
                                  sems.at[b]).wait()
        pltpu.sync_copy(acc.at[pl.ds(r0, WB)], bufs.at[b])
        pltpu.async_copy(bufs.at[b], out_hbm.at[c].at[pl.ds(r0, WB)], sems.at[b])
    for k in (NWB - 2, NWB - 1):
        b = k % 2
        r0 = s * NPS + k * WB
        pltpu.make_async_copy(bufs.at[b], out_hbm.at[c].at[pl.ds(r0, WB)],
                              sems.at[b]).wait()


@functools.partial(
    pl.kernel,
    out_type=jax.ShapeDtypeStruct((NC, NPAD, HID), _f32),
    mesh=_mesh,
    compiler_params=pltpu.CompilerParams(use_tc_tiling_on_sc=False),
    scratch_types=[
        pltpu.VMEM((CPW + 1, CH), jnp.int32),
        pltpu.VMEM((CPW + 1, CH), jnp.int32),
        pltpu.VMEM((RB, CH, HID), _f32),
        pltpu.VMEM((NPS, HID), _f32),
        pltpu.VMEM((WB, HID), _f32),
        pltpu.VMEM_SHARED((NPAD, HID), _f32),
        pltpu.VMEM_SHARED((NPAD, HID), _f32),
        pltpu.SemaphoreType.DMA((RB,)),
        pltpu.SemaphoreType.DMA((RB,)),
    ],
)
def _sc_agg(g_hbm, ei_hbm, out_hbm, sidx, didx, rows, stg, buf,
            gs, acc, gsem, ssem):
    c = lax.axis_index("c")
    s = lax.axis_index("s")
    wid = c * NS + s
    tbase = wid * CPW + jnp.minimum(wid, XTRA)
    extra = wid < XTRA
    t0 = s * NPS

    cp_si = pltpu.async_copy(ei_hbm.at[0].at[pl.ds(tbase, CPW)],
                             sidx.at[pl.ds(0, CPW)], gsem.at[0])
    cp_di = pltpu.async_copy(ei_hbm.at[1].at[pl.ds(tbase, CPW)],
                             didx.at[pl.ds(0, CPW)], gsem.at[1])
    cp_st = pltpu.async_copy(g_hbm.at[pl.ds(t0, NPS)], stg, ssem.at[0])

    @pl.when(extra)
    def _():
        pltpu.async_copy(ei_hbm.at[0].at[pl.ds(tbase + CPW, 1)],
                         sidx.at[pl.ds(CPW, 1)], gsem.at[2])
        pltpu.async_copy(ei_hbm.at[1].at[pl.ds(tbase + CPW, 1)],
                         didx.at[pl.ds(CPW, 1)], gsem.at[3])

    _zero_fill(buf, WB)
    for k in range(NWB):
        pltpu.async_copy(buf, acc.at[pl.ds(t0 + k * WB, WB)], ssem.at[1])
    cp_st.wait()
    cp_gs = pltpu.async_copy(stg, gs.at[pl.ds(t0, NPS)], ssem.at[2])
    cp_si.wait()
    cp_di.wait()

    @pl.when(extra)
    def _():
        pltpu.make_async_copy(ei_hbm.at[0].at[pl.ds(tbase + CPW, 1)],
                              sidx.at[pl.ds(CPW, 1)], gsem.at[2]).wait()
        pltpu.make_async_copy(ei_hbm.at[1].at[pl.ds(tbase + CPW, 1)],
                              didx.at[pl.ds(CPW, 1)], gsem.at[3]).wait()

    for k in range(NWB):
        pltpu.make_async_copy(buf, acc.at[pl.ds(t0 + k * WB, WB)],
                              ssem.at[1]).wait()
    cp_gs.wait()
    plsc.subcore_barrier()

    def start_g(j, b):
        pltpu.async_copy(gs.at[sidx.at[j]], rows.at[b], gsem.at[b])

    def wait_g(j, b):
        pltpu.make_async_copy(gs.at[sidx.at[j]], rows.at[b], gsem.at[b]).wait()

    def start_s(j, b):
        pltpu.async_copy(rows.at[b], acc.at[didx.at[j]], ssem.at[b], add=True)

    def wait_s(j, b):
        pltpu.make_async_copy(rows.at[b], acc.at[didx.at[j]], ssem.at[b]).wait()

    NG = CPW // RB
    TAIL = CPW - RB * NG
    assert RB * NG >= RB and TAIL + 1 <= RB
    for b in range(RB):
        start_g(b, b)

    def body(q, carry):
        for b in range(RB):
            j = RB * q + b
            wait_g(j, b)
            start_s(j, b)
        for b in range(RB):
            j = RB * q + b
            wait_s(j, b)
            start_g(j + RB, b)
        return carry

    lax.fori_loop(0, NG - 1, body, 0)
    for b in range(RB):
        j = RB * (NG - 1) + b
        wait_g(j, b)
        start_s(j, b)
    j0 = RB * NG
    for t in range(TAIL):
        wait_s(j0 - RB + t, t)
        start_g(j0 + t, t)
    wait_s(j0 - RB + TAIL, TAIL)

    @pl.when(extra)
    def _():
        start_g(CPW, TAIL)

    for b in range(TAIL + 1, RB):
        wait_s(j0 - RB + b, b)
    for t in range(TAIL):
        wait_g(j0 + t, t)
        start_s(j0 + t, t)

    @pl.when(extra)
    def _():
        wait_g(CPW, TAIL)
        start_s(CPW, TAIL)
        wait_s(CPW, TAIL)

    for t in range(TAIL):
        wait_s(j0 + t, t)
    plsc.subcore_barrier()

    def bounce(b):
        return rows.at[b].at[pl.ds(0, WB)]

    for k in range(NWB):
        b = k % 2
        r0 = s * NPS + k * WB
        if k >= 2:
            rp = s * NPS + (k - 2) * WB
            pltpu.make_async_copy(bounce(b), out_hbm.at[c].at[pl.ds(rp, WB)],
                                  gsem.at[b]).wait()
        pltpu.sync_copy(acc.at[pl.ds(r0, WB)], bounce(b))
        pltpu.async_copy(bounce(b), out_hbm.at[c].at[pl.ds(r0, WB)], gsem.at[b])
    for k in (NWB - 2, NWB - 1):
        b = k % 2
        r0 = s * NPS + k * WB
        pltpu.make_async_copy(bounce(b), out_hbm.at[c].at[pl.ds(r0, WB)],
                              gsem.at[b]).wait()


RF = NPAD * HID // 128
RN = N * HID // 128


def _tc_h1_body(x2_ref, w1b_ref, h_ref):
    h_ref[...] = jnp.dot(x2_ref[...], w1b_ref[...], preferred_element_type=_f32)


def _tc_g1_body(h_ref, degp_ref, dis_ref, g1_ref):
    dis = lax.rsqrt(degp_ref[0] + degp_ref[1] + 1.0)
    dis_ref[...] = dis
    g1_ref[:RN] = h_ref[...] * dis[:RN]
    g1_ref[RN:] = jnp.zeros((RF - RN, 128), _f32)


def _tc_g2_body(aggp_ref, g1_ref, dis_ref, b1_ref, g2_ref):
    dis = dis_ref[...]
    t = dis * (aggp_ref[0] + aggp_ref[1] + g1_ref[...]) + b1_ref[...]
    g2_ref[...] = jnp.maximum(t, 0.0) * dis


def _tc_out_body(aggp_ref, g2_ref, dis_ref, w2b_ref, b2_ref, o_ref):
    t = dis_ref[:RN] * (aggp_ref[0, :RN] + aggp_ref[1, :RN] + g2_ref[:RN])
    o_ref[...] = jnp.dot(t, w2b_ref[...], preferred_element_type=_f32) + b2_ref[...]


def kernel(x, edge_index, W1, b1, W2, b2):
    ei2 = edge_index.reshape(2, NCHT, CH)

    x2 = x.reshape(RN, F_IN * N // RN)
    w1b = jnp.kron(jnp.eye(8, dtype=_f32), W1)
    w2b = jnp.kron(jnp.eye(8, dtype=_f32), W2)
    b1b = jnp.tile(b1, 8).reshape(1, 128)
    b2b = jnp.tile(b2, 8).reshape(1, 8 * CLS)

    h_f = pl.pallas_call(
        _tc_h1_body,
        out_shape=jax.ShapeDtypeStruct((RN, 128), _f32),
    )(x2, w1b)

    degp = _sc_deg(ei2)

    dis_f, g1_f = pl.pallas_call(
        _tc_g1_body,
        out_shape=[
            jax.ShapeDtypeStruct((RF, 128), _f32),
            jax.ShapeDtypeStruct((RF, 128), _f32),
        ],
    )(h_f, degp.reshape(NC, RF, 128))

    aggp1 = _sc_agg(g1_f.reshape(NPAD, HID), ei2)

    g2_f = pl.pallas_call(
        _tc_g2_body,
        out_shape=jax.ShapeDtypeStruct((RF, 128), _f32),
    )(aggp1.reshape(NC, RF, 128), g1_f, dis_f, b1b)

    aggp2 = _sc_agg(g2_f.reshape(NPAD, HID), ei2)

    out_f = pl.pallas_call(
        _tc_out_body,
        out_shape=jax.ShapeDtypeStruct((RN, 8 * CLS), _f32),
    )(aggp2.reshape(NC, RF, 128), g2_f, dis_f, w2b, b2b)

    return out_f.reshape(N, CLS)


# speedup vs baseline: 109.1694x; 1.0375x over previous
import functools

import jax
import jax.numpy as jnp
from jax import lax
from jax.experimental import pallas as pl
from jax.experimental.pallas import tpu as pltpu
from jax.experimental.pallas import tpu_sc as plsc

N = 10000
E = 320000
F_IN = 128
HID = 16
CLS = 40

NC = 2
NS = 16
L = 16

NW = NC * NS
CH = 512
NCHT = E // CH
CPW = NCHT // NW
XTRA = NCHT - NW * CPW
NPAD = 10240
NPS = NPAD // NS
RB = 8
WB = 128
NWB = NPS // WB

_mesh = plsc.VectorSubcoreMesh(core_axis_name="c", subcore_axis_name="s")

_f32 = jnp.float32


def _zero_fill(buf, nrows):
    zero = jnp.zeros((L,), _f32)

    def fill(i, carry):
        buf[i, :] = zero
        return carry

    lax.fori_loop(0, nrows, fill, 0)


def _load_chunks(ei_hbm, row, tbase, extra, idx):
    pltpu.sync_copy(ei_hbm.at[row].at[pl.ds(tbase, CPW)], idx.at[pl.ds(0, CPW)])

    @pl.when(extra)
    def _():
        pltpu.sync_copy(ei_hbm.at[row].at[pl.ds(tbase + CPW, 1)],
                        idx.at[pl.ds(CPW, 1)])


@functools.partial(
    pl.kernel,
    out_type=jax.ShapeDtypeStruct((NC, NPAD, HID), _f32),
    mesh=_mesh,
    compiler_params=pltpu.CompilerParams(use_tc_tiling_on_sc=False),
    scratch_types=[
        pltpu.VMEM((CPW + 1, CH), jnp.int32),
        pltpu.VMEM((CH, HID), _f32),
        pltpu.VMEM((2, WB, HID), _f32),
        pltpu.VMEM_SHARED((NPAD, HID), _f32),
        pltpu.SemaphoreType.DMA((4,)),
    ],
)
def _sc_deg(ei_hbm, out_hbm, didx, ones_b, bufs, acc, sems):
    c = lax.axis_index("c")
    s = lax.axis_index("s")
    wid = c * NS + s
    tbase = wid * CPW + jnp.minimum(wid, XTRA)
    extra = wid < XTRA
    sem = sems.at[3]
    buf = bufs.at[0]

    cp_di = pltpu.async_copy(ei_hbm.at[1].at[pl.ds(tbase, CPW)],
                             didx.at[pl.ds(0, CPW)], sems.at[2])

    @pl.when(extra)
    def _():
        pltpu.async_copy(ei_hbm.at[1].at[pl.ds(tbase + CPW, 1)],
                         didx.at[pl.ds(CPW, 1)], sems.at[1])

    _zero_fill(buf, WB)
    one = jnp.ones((L,), _f32)

    def fill_ones(i, carry):
        ones_b[i, :] = one
        return carry

    lax.fori_loop(0, CH, fill_ones, 0)

    for k in range(NWB):
        pltpu.async_copy(buf, acc.at[pl.ds(s * NPS + k * WB, WB)], sems.at[0])
    cp_di.wait()

    @pl.when(extra)
    def _():
        pltpu.make_async_copy(ei_hbm.at[1].at[pl.ds(tbase + CPW, 1)],
                              didx.at[pl.ds(CPW, 1)], sems.at[1]).wait()

    for k in range(NWB):
        pltpu.make_async_copy(buf, acc.at[pl.ds(s * NPS + k * WB, WB)],
                              sems.at[0]).wait()
    plsc.subcore_barrier()

    DW = 8

    def body(j, carry):
        pltpu.async_copy(ones_b, acc.at[didx.at[j]], sem, add=True)

        @pl.when(j >= DW)
        def _():
            pltpu.make_async_copy(ones_b, acc.at[didx.at[j - DW]], sem).wait()

        return carry

    lax.fori_loop(0, CPW, body, 0)
    for k in range(DW):
        pltpu.make_async_copy(ones_b, acc.at[didx.at[CPW - DW + k]], sem).wait()

    @pl.when(extra)
    def _():
        pltpu.sync_copy(ones_b, acc.at[didx.at[CPW]], add=True)

    plsc.subcore_barrier()

    for k in range(NWB):
        b = k % 2
        r0 = s * NPS + k * WB
        if k >= 2:
            rp = s * NPS + (k - 2) * WB
            pltpu.make_async_copy(bufs.at[b], out_hbm.at[c].at[pl.ds(rp, WB)],
                                  sems.at[b]).wait()
        pltpu.sync_copy(acc.at[pl.ds(r0, WB)], bufs.at[b])
        pltpu.async_copy(bufs.at[b], out_hbm.at[c].at[pl.ds(r0, WB)], sems.at[b])
    for k in (NWB - 2, NWB - 1):
        b = k % 2
        r0 = s * NPS + k * WB
        pltpu.make_async_copy(bufs.at[b], out_hbm.at[c].at[pl.ds(r0, WB)],
                              sems.at[b]).wait()


@functools.partial(
    pl.kernel,
    out_type=jax.ShapeDtypeStruct((NC, NPAD, HID), _f32),
    mesh=_mesh,
    compiler_params=pltpu.CompilerParams(use_tc_tiling_on_sc=False),
    scratch_types=[
        pltpu.VMEM((CPW + 1, CH), jnp.int32),
        pltpu.VMEM((CPW + 1, CH), jnp.int32),
        pltpu.VMEM((RB, CH, HID), _f32),
        pltpu.VMEM((NPS, HID), _f32),
        pltpu.VMEM((WB, HID), _f32),
        pltpu.VMEM_SHARED((NPAD, HID), _f32),
        pltpu.VMEM_SHARED((NPAD, HID), _f32),
        pltpu.SemaphoreType.DMA((RB,)),
        pltpu.SemaphoreType.DMA((RB,)),
    ],
)
def _sc_agg(g_hbm, ei_hbm, out_hbm, sidx, didx, rows, stg, buf,
            gs, acc, gsem, ssem):
    c = lax.axis_index("c")
    s = lax.axis_index("s")
    wid = c * NS + s
    tbase = wid * CPW + jnp.minimum(wid, XTRA)
    extra = wid < XTRA
    t0 = s * NPS

    cp_si = pltpu.async_copy(ei_hbm.at[0].at[pl.ds(tbase, CPW)],
                             sidx.at[pl.ds(0, CPW)], gsem.at[0])
    cp_di = pltpu.async_copy(ei_hbm.at[1].at[pl.ds(tbase, CPW)],
                             didx.at[pl.ds(0, CPW)], gsem.at[1])
    cp_st = pltpu.async_copy(g_hbm.at[pl.ds(t0, NPS)], stg, ssem.at[0])

    @pl.when(extra)
    def _():
        pltpu.async_copy(ei_hbm.at[0].at[pl.ds(tbase + CPW, 1)],
                         sidx.at[pl.ds(CPW, 1)], gsem.at[2])
        pltpu.async_copy(ei_hbm.at[1].at[pl.ds(tbase + CPW, 1)],
                         didx.at[pl.ds(CPW, 1)], gsem.at[3])

    _zero_fill(buf, WB)
    for k in range(NWB):
        pltpu.async_copy(buf, acc.at[pl.ds(t0 + k * WB, WB)], ssem.at[1])
    cp_st.wait()
    cp_gs = pltpu.async_copy(stg, gs.at[pl.ds(t0, NPS)], ssem.at[2])
    cp_si.wait()
    cp_di.wait()

    @pl.when(extra)
    def _():
        pltpu.make_async_copy(ei_hbm.at[0].at[pl.ds(tbase + CPW, 1)],
                              sidx.at[pl.ds(CPW, 1)], gsem.at[2]).wait()
        pltpu.make_async_copy(ei_hbm.at[1].at[pl.ds(tbase + CPW, 1)],
                              didx.at[pl.ds(CPW, 1)], gsem.at[3]).wait()

    for k in range(NWB):
        pltpu.make_async_copy(buf, acc.at[pl.ds(t0 + k * WB, WB)],
                              ssem.at[1]).wait()
    cp_gs.wait()
    plsc.subcore_barrier()

    def start_g(j, b):
        pltpu.async_copy(gs.at[sidx.at[j]], rows.at[b], gsem.at[b])

    def wait_g(j, b):
        pltpu.make_async_copy(gs.at[sidx.at[j]], rows.at[b], gsem.at[b]).wait()

    def start_s(j, b):
        pltpu.async_copy(rows.at[b], acc.at[didx.at[j]], ssem.at[b], add=True)

    def wait_s(j, b):
        pltpu.make_async_copy(rows.at[b], acc.at[didx.at[j]], ssem.at[b]).wait()

    NG = CPW // RB
    TAIL = CPW - RB * NG
    assert RB * NG >= RB and TAIL + 1 <= RB
    for b in range(RB):
        start_g(b, b)

    def body(q, carry):
        for b in range(RB):
            j = RB * q + b
            wait_g(j, b)
            start_s(j, b)
        for b in range(RB):
            j = RB * q + b
            wait_s(j, b)
            start_g(j + RB, b)
        return carry

    lax.fori_loop(0, NG - 1, body, 0)
    for b in range(RB):
        j = RB * (NG - 1) + b
        wait_g(j, b)
        start_s(j, b)
    j0 = RB * NG
    for t in range(TAIL):
        wait_s(j0 - RB + t, t)
        start_g(j0 + t, t)
    wait_s(j0 - RB + TAIL, TAIL)

    @pl.when(extra)
    def _():
        start_g(CPW, TAIL)

    for b in range(TAIL + 1, RB):
        wait_s(j0 - RB + b, b)
    for t in range(TAIL):
        wait_g(j0 + t, t)
        start_s(j0 + t, t)

    @pl.when(extra)
    def _():
        wait_g(CPW, TAIL)
        start_s(CPW, TAIL)
        wait_s(CPW, TAIL)

    for t in range(TAIL):
        wait_s(j0 + t, t)
    plsc.subcore_barrier()

    def bounce(b):
        return rows.at[b].at[pl.ds(0, WB)]

    for k in range(NWB):
        b = k % 2
        r0 = s * NPS + k * WB
        if k >= 2:
            rp = s * NPS + (k - 2) * WB
            pltpu.make_async_copy(bounce(b), out_hbm.at[c].at[pl.ds(rp, WB)],
                                  gsem.at[b]).wait()
        pltpu.sync_copy(acc.at[pl.ds(r0, WB)], bounce(b))
        pltpu.async_copy(bounce(b), out_hbm.at[c].at[pl.ds(r0, WB)], gsem.at[b])
    for k in (NWB - 2, NWB - 1):
        b = k % 2
        r0 = s * NPS + k * WB
        pltpu.make_async_copy(bounce(b), out_hbm.at[c].at[pl.ds(r0, WB)],
                              gsem.at[b]).wait()


RF = NPAD * HID // 128
RN = N * HID // 128


def _tc_h1_body(x2_ref, w1b_ref, h_ref):
    h_ref[...] = jnp.dot(x2_ref[...], w1b_ref[...], preferred_element_type=_f32)


def _tc_g1_body(h_ref, degp_ref, dis_ref, g1_ref):
    dis = lax.rsqrt(degp_ref[0] + degp_ref[1] + 1.0)
    dis_ref[...] = dis
    g1_ref[:RN] = h_ref[...] * dis[:RN]
    g1_ref[RN:] = jnp.zeros((RF - RN, 128), _f32)


def _tc_g2_body(aggp_ref, g1_ref, dis_ref, b1_ref, g2_ref):
    dis = dis_ref[...]
    t = dis * (aggp_ref[0] + aggp_ref[1] + g1_ref[...]) + b1_ref[...]
    g2_ref[...] = jnp.maximum(t, 0.0) * dis


def _tc_out_body(aggp_ref, g2_ref, dis_ref, w2b_ref, b2_ref, o_ref):
    t = dis_ref[:RN] * (aggp_ref[0, :RN] + aggp_ref[1, :RN] + g2_ref[:RN])
    o_ref[...] = jnp.dot(t, w2b_ref[...], preferred_element_type=_f32) + b2_ref[...]


def kernel(x, edge_index, W1, b1, W2, b2):
    ei2 = edge_index.reshape(2, NCHT, CH)

    x2 = x.reshape(RN, F_IN * N // RN)
    w1b = jnp.kron(jnp.eye(8, dtype=_f32), W1)
    w2b = jnp.kron(jnp.eye(8, dtype=_f32), W2)
    b1b = jnp.tile(b1, 8).reshape(1, 128)
    b2b = jnp.tile(b2, 8).reshape(1, 8 * CLS)

    h_f = pl.pallas_call(
        _tc_h1_body,
        out_shape=jax.ShapeDtypeStruct((RN, 128), _f32),
    )(x2, w1b)

    degp = _sc_deg(ei2)

    dis_f, g1_f = pl.pallas_call(
        _tc_g1_body,
        out_shape=[
            jax.ShapeDtypeStruct((RF, 128), _f32),
            jax.ShapeDtypeStruct((RF, 128), _f32),
        ],
    )(h_f, degp.reshape(NC, RF, 128))

    aggp1 = _sc_agg(g1_f.reshape(NPAD, HID), ei2)

    g2_f = pl.pallas_call(
        _tc_g2_body,
        out_shape=jax.ShapeDtypeStruct((RF, 128), _f32),
    )(aggp1.reshape(NC, RF, 128), g1_f, dis_f, b1b)

    aggp2 = _sc_agg(g2_f.reshape(NPAD, HID), ei2)

    out_f = pl.pallas_call(
        _tc_out_body,
        out_shape=jax.ShapeDtypeStruct((RN, 8 * CLS), _f32),
    )(aggp2.reshape(NC, RF, 128), g2_f, dis_f, w2b, b2b)

    return out_f.reshape(N, CLS)
